# baseline (speedup 1.0000x reference)
import functools

import jax
import jax.numpy as jnp
from jax import lax
from jax.experimental import pallas as pl
from jax.experimental.pallas import tpu as pltpu
from jax.experimental.pallas import tpu_sc as plsc

B, N = 64, 8192
TOP_N = 500
L = 16
NV = N // L
NB1 = 8192
NB2 = 1024
NB3 = 256
U = 16
NW = 32
ROWS_PER_W = 2

_mesh = plsc.VectorSubcoreMesh(core_axis_name="c", subcore_axis_name="s")


@functools.partial(
    pl.kernel,
    mesh=_mesh,
    out_type=jax.ShapeDtypeStruct((B, N), jnp.float32),
    compiler_params=pltpu.CompilerParams(needs_layout_passes=False),
    scratch_types=[
        pltpu.VMEM((N,), jnp.float32),
        pltpu.VMEM((N,), jnp.float32),
        pltpu.VMEM((N,), jnp.float32),
        pltpu.VMEM((N,), jnp.float32),
        pltpu.VMEM((N,), jnp.float32),
        pltpu.VMEM((N,), jnp.float32),
        pltpu.VMEM((NB1,), jnp.int32),
        pltpu.VMEM((NB2,), jnp.int32),
        pltpu.VMEM((NB3,), jnp.int32),
        pltpu.VMEM((L,), jnp.float32),
        pltpu.SemaphoreType.DMA,
        pltpu.SemaphoreType.DMA,
        pltpu.SemaphoreType.DMA,
    ],
)
def _sc_topk_mask(diff_hbm, cond_hbm, weight_hbm, out_hbm,
                  d0_ref, d1_ref, w0_ref, w1_ref, o0_ref, o1_ref,
                  h1_ref, h2_ref, h3_ref, cond_ref,
                  d_sem, w_sem, o_sem):
    wid = lax.axis_index("c") * 16 + lax.axis_index("s")
    row0 = wid * ROWS_PER_W
    d_refs = [d0_ref, d1_ref]
    w_refs = [w0_ref, w1_ref]
    o_refs = [o0_ref, o1_ref]

    d_cp = [pltpu.async_copy(diff_hbm.at[row0 + r], d_refs[r], d_sem)
            for r in range(ROWS_PER_W)]
    w_cp = [pltpu.async_copy(weight_hbm.at[row0 + r], w_refs[r], w_sem)
            for r in range(ROWS_PER_W)]
    pltpu.sync_copy(cond_hbm.at[wid], cond_ref)
    condv = cond_ref[...]
    zeros_f = jnp.zeros((L,), jnp.float32)
    iota = lax.iota(jnp.int32, L)
    ones = jnp.ones((L,), jnp.int32)
    zeros = jnp.zeros((L,), jnp.int32)

    def _zero(ref, n):
        @plsc.parallel_loop(0, n // L, unroll=U)
        def _(j):
            ref[pl.ds(j * L, L)] = zeros

    def _scan(load, ngroups, start_bucket, target):
        def cond(c):
            j, cum, E, need, cnt = c
            return (E < 0) & (j >= 0)

        def body(c):
            j, cum, E, need, cnt = c
            h = load(j)
            hd = lax.rev(h, (0,))
            inc = jnp.cumsum(hd)
            crossed = (cum + inc) >= target
            lane = jnp.min(jnp.where(crossed, iota, L))
            found = lane < L
            inc_l = jnp.sum(jnp.where(iota == lane, inc, 0))
            hd_l = jnp.sum(jnp.where(iota == lane, hd, 0))
            E = jnp.where(found, j * L + (L - 1) - lane, E)
            need = jnp.where(found, target - (cum + inc_l - hd_l), need)
            cnt = jnp.where(found, hd_l, cnt)
            return j - 1, cum + jnp.sum(h), E, need, cnt

        _, _, E, need, cnt = lax.while_loop(
            cond, body,
            (start_bucket >> 4, jnp.int32(0), jnp.int32(-1), jnp.int32(0),
             jnp.int32(0)))
        return E, need, cnt

    for r in range(ROWS_PER_W):
        d_cp[r].wait()
        dr = d_refs[r]

        _zero(h1_ref, NB1)

        def _p1(i, mx):
            v = dr[pl.ds(i * L, L)]
            b = lax.bitcast_convert_type(v, jnp.int32)
            bk = lax.shift_right_logical(b, 18) & 0x1FFF
            plsc.addupdate_scatter(h1_ref, [bk], ones)
            return jnp.maximum(mx, bk)
        mx1 = plsc.parallel_loop(0, NV, unroll=U, carry=zeros)(_p1)

        def _ld1(j):
            return h1_ref[pl.ds(j * L, L)]
        E1, need1, _ = _scan(_ld1, NB1 // L, jnp.max(mx1), jnp.int32(TOP_N))

        _zero(h2_ref, NB2)

        def _p2(i, mx):
            v = dr[pl.ds(i * L, L)]
            b = lax.bitcast_convert_type(v, jnp.int32)
            m = (lax.shift_right_logical(b, 18) & 0x1FFF) == E1
            bk = (b >> 8) & 0x3FF
            plsc.addupdate_scatter(h2_ref, [bk], ones, mask=m)
            return jnp.maximum(mx, jnp.where(m, bk, 0))
        mx2 = plsc.parallel_loop(0, NV, unroll=U, carry=zeros)(_p2)

        def _ld2(j):
            return h2_ref[pl.ds(j * L, L)]
        E2, need2, _ = _scan(_ld2, NB2 // L, jnp.max(mx2), need1)
        P = (E1 << 10) | E2

        _zero(h3_ref, NB3)

        def _p3(i, mx):
            v = dr[pl.ds(i * L, L)]
            b = lax.bitcast_convert_type(v, jnp.int32)
            m = (lax.shift_right_logical(b, 8) & 0x7FFFFF) == P
            bk = b & 0xFF
            plsc.addupdate_scatter(h3_ref, [bk], ones, mask=m)
            return jnp.maximum(mx, jnp.where(m, bk, 0))
        mx3 = plsc.parallel_loop(0, NV, unroll=U, carry=zeros)(_p3)

        def _ld3(j):
            return h3_ref[pl.ds(j * L, L)]
        E3, need_eq, cnt_eq = _scan(_ld3, NB3 // L, jnp.max(mx3), need2)
        t = (P << 8) | E3
        need_drop = cnt_eq - need_eq

        w_cp[r].wait()
        wr = w_refs[r]
        orr = o_refs[r]

        @plsc.parallel_loop(0, NV, unroll=U)
        def _(i):
            v = dr[pl.ds(i * L, L)]
            b = lax.bitcast_convert_type(v, jnp.int32) & 0x7FFFFFFF
            wv = wr[pl.ds(i * L, L)]
            orr[pl.ds(i * L, L)] = wv + jnp.where(b >= t, condv, zeros_f)

        @pl.when(need_drop > 0)
        def _tie_fix():
            def _fix(jj, run_end):
                i = NV - 1 - jj
                v = dr[pl.ds(i * L, L)]
                b = lax.bitcast_convert_type(v, jnp.int32) & 0x7FFFFFFF
                eq = b == t
                eqi = eq.astype(jnp.int32)
                rank_end = lax.rev(jnp.cumsum(lax.rev(eqi, (0,))), (0,))
                drop = eq & ((run_end + rank_end) <= need_drop)
                ov = orr[pl.ds(i * L, L)]
                orr[pl.ds(i * L, L)] = ov - jnp.where(drop, condv, zeros_f)
                return run_end + jnp.sum(eqi)
            lax.fori_loop(0, NV, _fix, jnp.int32(0), unroll=False)

        pltpu.async_copy(orr, out_hbm.at[row0 + r], o_sem)

    for r in range(ROWS_PER_W):
        pltpu.make_async_copy(o_refs[r], out_hbm.at[row0 + r], o_sem).wait()


def kernel(difference, weight, epoch, iteration):
    cond = (epoch > 1) & (epoch % 2 == 0)
    condf = jnp.where(cond, jnp.float32(1.0), jnp.float32(0.0))
    cond_rows = jnp.broadcast_to(condf, (NW, L))
    return _sc_topk_mask(difference, cond_rows, weight)


# speedup vs baseline: 1.0242x; 1.0242x over previous
import functools

import jax
import jax.numpy as jnp
from jax import lax
from jax.experimental import pallas as pl
from jax.experimental.pallas import tpu as pltpu
from jax.experimental.pallas import tpu_sc as plsc

B, N = 64, 8192
TOP_N = 500
L = 16
NV = N // L
NB1 = 8192
NB2 = 1024
NB3 = 256
U = 8
NW = 32
ROWS_PER_W = 2

_mesh = plsc.VectorSubcoreMesh(core_axis_name="c", subcore_axis_name="s")


@functools.partial(
    pl.kernel,
    mesh=_mesh,
    out_type=jax.ShapeDtypeStruct((B, N), jnp.float32),
    compiler_params=pltpu.CompilerParams(needs_layout_passes=False),
    scratch_types=[
        pltpu.VMEM((N,), jnp.float32),
        pltpu.VMEM((N,), jnp.float32),
        pltpu.VMEM((N,), jnp.float32),
        pltpu.VMEM((N,), jnp.float32),
        pltpu.VMEM((N,), jnp.float32),
        pltpu.VMEM((N,), jnp.float32),
        pltpu.VMEM((NB1,), jnp.int32),
        pltpu.VMEM((NB2,), jnp.int32),
        pltpu.VMEM((NB3,), jnp.int32),
        pltpu.VMEM((L,), jnp.float32),
        pltpu.SemaphoreType.DMA,
        pltpu.SemaphoreType.DMA,
        pltpu.SemaphoreType.DMA,
    ],
)
def _sc_topk_mask(diff_hbm, cond_hbm, weight_hbm, out_hbm,
                  d0_ref, d1_ref, w0_ref, w1_ref, o0_ref, o1_ref,
                  h1_ref, h2_ref, h3_ref, cond_ref,
                  d_sem, w_sem, o_sem):
    wid = lax.axis_index("c") * 16 + lax.axis_index("s")
    row0 = wid * ROWS_PER_W
    d_refs = [d0_ref, d1_ref]
    w_refs = [w0_ref, w1_ref]
    o_refs = [o0_ref, o1_ref]

    d_cp = [pltpu.async_copy(diff_hbm.at[row0 + r], d_refs[r], d_sem)
            for r in range(ROWS_PER_W)]
    w_cp = [pltpu.async_copy(weight_hbm.at[row0 + r], w_refs[r], w_sem)
            for r in range(ROWS_PER_W)]
    pltpu.sync_copy(cond_hbm.at[wid], cond_ref)
    condv = cond_ref[...]
    zeros_f = jnp.zeros((L,), jnp.float32)
    iota = lax.iota(jnp.int32, L)
    ones = jnp.ones((L,), jnp.int32)
    zeros = jnp.zeros((L,), jnp.int32)

    def _zero(ref, n):
        @plsc.parallel_loop(0, n // L, unroll=U)
        def _(j):
            ref[pl.ds(j * L, L)] = zeros

    def _scan(load, ngroups, start_bucket, target):
        def cond(c):
            j, cum, E, need, cnt = c
            return (E < 0) & (j >= 0)

        def body(c):
            j, cum, E, need, cnt = c
            h = load(j)
            hd = lax.rev(h, (0,))
            inc = jnp.cumsum(hd)
            crossed = (cum + inc) >= target
            lane = jnp.min(jnp.where(crossed, iota, L))
            found = lane < L
            inc_l = jnp.sum(jnp.where(iota == lane, inc, 0))
            hd_l = jnp.sum(jnp.where(iota == lane, hd, 0))
            E = jnp.where(found, j * L + (L - 1) - lane, E)
            need = jnp.where(found, target - (cum + inc_l - hd_l), need)
            cnt = jnp.where(found, hd_l, cnt)
            return j - 1, cum + jnp.sum(h), E, need, cnt

        _, _, E, need, cnt = lax.while_loop(
            cond, body,
            (start_bucket >> 4, jnp.int32(0), jnp.int32(-1), jnp.int32(0),
             jnp.int32(0)))
        return E, need, cnt

    for r in range(ROWS_PER_W):
        d_cp[r].wait()
        dr = d_refs[r]

        _zero(h1_ref, NB1)

        def _p1(i, mx):
            v = dr[pl.ds(i * L, L)]
            b = lax.bitcast_convert_type(v, jnp.int32)
            bk = lax.shift_right_logical(b, 18) & 0x1FFF
            plsc.addupdate_scatter(h1_ref, [bk], ones)
            return jnp.maximum(mx, bk)
        mx1 = plsc.parallel_loop(0, NV, unroll=U, carry=zeros)(_p1)

        def _ld1(j):
            return h1_ref[pl.ds(j * L, L)]
        E1, need1, _ = _scan(_ld1, NB1 // L, jnp.max(mx1), jnp.int32(TOP_N))

        _zero(h2_ref, NB2)

        def _p2(i, mx):
            v = dr[pl.ds(i * L, L)]
            b = lax.bitcast_convert_type(v, jnp.int32)
            m = (lax.shift_right_logical(b, 18) & 0x1FFF) == E1
            bk = (b >> 8) & 0x3FF
            plsc.addupdate_scatter(h2_ref, [bk], ones, mask=m)
            return jnp.maximum(mx, jnp.where(m, bk, 0))
        mx2 = plsc.parallel_loop(0, NV, unroll=U, carry=zeros)(_p2)

        def _ld2(j):
            return h2_ref[pl.ds(j * L, L)]
        E2, need2, _ = _scan(_ld2, NB2 // L, jnp.max(mx2), need1)
        P = (E1 << 10) | E2

        _zero(h3_ref, NB3)

        def _p3(i, mx):
            v = dr[pl.ds(i * L, L)]
            b = lax.bitcast_convert_type(v, jnp.int32)
            m = (lax.shift_right_logical(b, 8) & 0x7FFFFF) == P
            bk = b & 0xFF
            plsc.addupdate_scatter(h3_ref, [bk], ones, mask=m)
            return jnp.maximum(mx, jnp.where(m, bk, 0))
        mx3 = plsc.parallel_loop(0, NV, unroll=U, carry=zeros)(_p3)

        def _ld3(j):
            return h3_ref[pl.ds(j * L, L)]
        E3, need_eq, cnt_eq = _scan(_ld3, NB3 // L, jnp.max(mx3), need2)
        t = (P << 8) | E3
        need_drop = cnt_eq - need_eq

        w_cp[r].wait()
        wr = w_refs[r]
        orr = o_refs[r]

        @plsc.parallel_loop(0, NV, unroll=U)
        def _(i):
            v = dr[pl.ds(i * L, L)]
            b = lax.bitcast_convert_type(v, jnp.int32) & 0x7FFFFFFF
            wv = wr[pl.ds(i * L, L)]
            orr[pl.ds(i * L, L)] = wv + jnp.where(b >= t, condv, zeros_f)

        @pl.when(need_drop > 0)
        def _tie_fix():
            def _fix(jj, run_end):
                i = NV - 1 - jj
                v = dr[pl.ds(i * L, L)]
                b = lax.bitcast_convert_type(v, jnp.int32) & 0x7FFFFFFF
                eq = b == t
                eqi = eq.astype(jnp.int32)
                rank_end = lax.rev(jnp.cumsum(lax.rev(eqi, (0,))), (0,))
                drop = eq & ((run_end + rank_end) <= need_drop)
                ov = orr[pl.ds(i * L, L)]
                orr[pl.ds(i * L, L)] = ov - jnp.where(drop, condv, zeros_f)
                return run_end + jnp.sum(eqi)
            lax.fori_loop(0, NV, _fix, jnp.int32(0), unroll=False)

        pltpu.async_copy(orr, out_hbm.at[row0 + r], o_sem)

    for r in range(ROWS_PER_W):
        pltpu.make_async_copy(o_refs[r], out_hbm.at[row0 + r], o_sem).wait()


def kernel(difference, weight, epoch, iteration):
    cond = (epoch > 1) & (epoch % 2 == 0)
    condf = jnp.where(cond, jnp.float32(1.0), jnp.float32(0.0))
    cond_rows = jnp.broadcast_to(condf, (NW, L))
    return _sc_topk_mask(difference, cond_rows, weight)
